# Initial kernel scaffold; baseline (speedup 1.0000x reference)
import numpy as np
import jax
import jax.numpy as jnp
from jax.experimental import pallas as pl

_COCO_SKELETON = [(0, 1), (0, 2), (1, 3), (2, 4), (5, 7), (7, 9), (6, 8), (8, 10), (5, 6), (5, 11), (6, 12), (11, 12), (11, 13), (13, 15), (12, 14), (14, 16)]
_LIMBS = [{0, 1, 2, 3, 4}, {5, 7, 9}, {6, 8, 10}, {5, 6, 11, 12}, {11, 13, 15}, {12, 14, 16}]
_NJT = 17
_N = 10000; _E = 320000; _IN_DIM = 128; _HID = 32; _HEADS = 4; _NSTD = 3; _NREP = 1; _OUT_DIM = 128


def _build_mats():
    skel = np.zeros((_NJT, _NJT), dtype=bool)
    for a, b in _COCO_SKELETON:
        skel[a, b] = True; skel[b, a] = True
    slimb = np.zeros((_NJT, _NJT), dtype=bool)
    for limb in _LIMBS:
        for a in limb:
            for b in limb:
                if a != b and not skel[a, b]:
                    slimb[a, b] = True
    return skel, slimb


_SKEL, _SLIMB = _build_mats()


def _ln(x, g, b):
    m = x.mean(-1, keepdims=True)
    v = ((x - m) ** 2).mean(-1, keepdims=True)
    return (x - m) / jnp.sqrt(v + 1e-5) * g + b


def _enc(f, p):
    h = jax.nn.relu(f @ p['W1'] + p['b1'])
    return h @ p['W2'] + p['b2']


def _gatv2_nomax(x, src, dst, ea, p, H, C, n, mask=None):
    xl = (x @ p['Wl'] + p['bl']).reshape(n, H, C)
    xr = (x @ p['Wr'] + p['br']).reshape(n, H, C)
    m = xl[src] + xr[dst] + (ea @ p['We']).reshape(-1, H, C)
    ma = jax.nn.leaky_relu(m, 0.2)
    alpha = (ma * p['att'][None]).sum(-1)
    ex = jnp.exp(alpha)
    if mask is not None:
        ex = jnp.where(mask[:, None], ex, 0.0)
    den = jax.ops.segment_sum(ex, dst, num_segments=n)
    num = jax.ops.segment_sum(xl[src] * ex[..., None], dst, num_segments=n)
    out = num / (den[..., None] + 1e-16)
    return out.reshape(n, H * C) + p['bias']


def _final_proj_kernel(h_ref, w_ref, b_ref, g_ref, beta_ref, o_ref):
    y = jnp.dot(h_ref[...], w_ref[...], preferred_element_type=jnp.float32)
    y = y + b_ref[...]
    m = y.mean(-1, keepdims=True)
    v = ((y - m) ** 2).mean(-1, keepdims=True)
    o_ref[...] = (y - m) / jnp.sqrt(v + 1e-5) * g_ref[...] + beta_ref[...]


def _final_proj(h, W, b, g, beta):
    n = h.shape[0]
    blk = 1000
    return pl.pallas_call(
        _final_proj_kernel,
        grid=(n // blk,),
        in_specs=[
            pl.BlockSpec((blk, h.shape[1]), lambda i: (i, 0)),
            pl.BlockSpec((W.shape[0], W.shape[1]), lambda i: (0, 0)),
            pl.BlockSpec((W.shape[1],), lambda i: (0,)),
            pl.BlockSpec((W.shape[1],), lambda i: (0,)),
            pl.BlockSpec((W.shape[1],), lambda i: (0,)),
        ],
        out_specs=pl.BlockSpec((blk, W.shape[1]), lambda i: (i, 0)),
        out_shape=jax.ShapeDtypeStruct((n, W.shape[1]), jnp.float32),
    )(h, W, b, g, beta)


def kernel(x, edge_index, joint_types, positions, params):
    src = edge_index[0]; dst = edge_index[1]
    ts = joint_types[src]; td = joint_types[dst]
    same = ts == td
    skel = jnp.asarray(_SKEL)[ts, td]
    lim = jnp.asarray(_SLIMB)[ts, td]
    cat = jnp.where(same, 0, jnp.where(skel, 1, jnp.where(lim, 2, 3)))
    onehot = jax.nn.one_hot(cat, 4, dtype=jnp.float32)
    rel = positions[dst] - positions[src]
    dist = jnp.sqrt((rel ** 2).sum(-1, keepdims=True) + 1e-12)
    eraw = jnp.concatenate([onehot, rel, dist], axis=1)
    h = x + params['emb'][joint_types]
    nl = len(params['layers'])
    for i in range(nl):
        lp = params['layers'][i]; npar = params['norms'][i]
        ea = _enc(eraw, lp['enc'])
        h_std = _gatv2_nomax(h, src, dst, ea, lp['std'], _NSTD, _HID, _N)
        h_rep = _gatv2_nomax(h, src, dst, ea, lp['rep'], _NREP, _HID, _N, mask=same)
        hcat = jnp.concatenate([h_std, h_rep], axis=-1)
        if i == nl - 1:
            hcat = hcat.reshape(_N, _HEADS, _HID).mean(1)
        h = jax.nn.elu(_ln(hcat, npar['g'], npar['b']))
    return _final_proj(h, params['proj_W'], params['proj_b'], params['final_g'], params['final_b'])


# speedup vs baseline: 11.1695x; 11.1695x over previous
import functools

import numpy as np
import jax
import jax.numpy as jnp
from jax import lax
from jax.experimental import pallas as pl
from jax.experimental.pallas import tpu as pltpu
from jax.experimental.pallas import tpu_sc as plsc

_COCO_SKELETON = [(0, 1), (0, 2), (1, 3), (2, 4), (5, 7), (7, 9), (6, 8), (8, 10), (5, 6), (5, 11), (6, 12), (11, 12), (11, 13), (13, 15), (12, 14), (14, 16)]
_LIMBS = [{0, 1, 2, 3, 4}, {5, 7, 9}, {6, 8, 10}, {5, 6, 11, 12}, {11, 13, 15}, {12, 14, 16}]
_NJT = 17
_N = 10000
_NPAD = 10240
_E = 320000
_NW = 32
_RPT = _NPAD // 16


_CHF = 128
_NCHUNKS = _E // _CHF
_CHE = 80
_EPW = _E // _NW
_NCHE = _EPW // _CHE
_DROWS = _NPAD * 4 // 128


def _build_mats():
    skel = np.zeros((_NJT, _NJT), dtype=bool)
    for a, b in _COCO_SKELETON:
        skel[a, b] = True; skel[b, a] = True
    slimb = np.zeros((_NJT, _NJT), dtype=bool)
    for limb in _LIMBS:
        for a in limb:
            for b in limb:
                if a != b and not skel[a, b]:
                    slimb[a, b] = True
    return skel, slimb


_SKEL_NP, _SLIMB_NP = _build_mats()
_SKEL_F = np.zeros((304,), np.float32); _SKEL_F[:289] = _SKEL_NP.astype(np.float32).reshape(-1)
_SLIMB_F = np.zeros((304,), np.float32); _SLIMB_F[:289] = _SLIMB_NP.astype(np.float32).reshape(-1)


def _sc_mesh():
    return plsc.VectorSubcoreMesh(core_axis_name="c", subcore_axis_name="s",
                                  num_cores=2, num_subcores=16)


def _splat(v, dtype=jnp.int32):
    return jnp.full((16,), v, dtype)


def _sc_edgefeat(src, dst, jt, px, py, skelf, limf):
    @functools.partial(
        pl.kernel, mesh=_sc_mesh(),
        compiler_params=pltpu.CompilerParams(needs_layout_passes=False),
        out_type=jax.ShapeDtypeStruct((8, _E), jnp.float32),
        scratch_types=[
            pltpu.VMEM((_CHF,), jnp.int32), pltpu.VMEM((_CHF,), jnp.int32),
            pltpu.VMEM((_N,), jnp.int32),
            pltpu.VMEM((_N,), jnp.float32), pltpu.VMEM((_N,), jnp.float32),
            pltpu.VMEM((304,), jnp.float32), pltpu.VMEM((304,), jnp.float32),
            pltpu.VMEM((8, _CHF), jnp.float32),
        ])
    def kfeat(src_h, dst_h, jt_h, px_h, py_h, skel_h, lim_h, out_h,
              v_src, v_dst, v_jt, v_px, v_py, v_sk, v_lm, v_st):
        cid = lax.axis_index("c"); sid = lax.axis_index("s")
        wid = sid * 2 + cid
        pltpu.sync_copy(jt_h, v_jt)
        pltpu.sync_copy(px_h, v_px)
        pltpu.sync_copy(py_h, v_py)
        pltpu.sync_copy(skel_h, v_sk)
        pltpu.sync_copy(lim_h, v_lm)
        rag = _NCHUNKS - (_NCHUNKS // _NW) * _NW
        nch = jnp.where(wid < rag, _NCHUNKS // _NW + 1, _NCHUNKS // _NW)

        def chunk(it, carry):
            base = (wid + _NW * it) * _CHF
            pltpu.sync_copy(src_h.at[pl.ds(base, _CHF)], v_src)
            pltpu.sync_copy(dst_h.at[pl.ds(base, _CHF)], v_dst)

            def group(g, c2):
                sl = pl.ds(g * 16, 16)
                s16 = v_src[sl]; d16 = v_dst[sl]
                jts = plsc.load_gather(v_jt, [s16])
                jtd = plsc.load_gather(v_jt, [d16])
                pxs = plsc.load_gather(v_px, [s16])
                pxd = plsc.load_gather(v_px, [d16])
                pys = plsc.load_gather(v_py, [s16])
                pyd = plsc.load_gather(v_py, [d16])
                ji = jts * 17 + jtd
                sk = plsc.load_gather(v_sk, [ji])
                lm = plsc.load_gather(v_lm, [ji])
                samef = jnp.where(jts == jtd, 1.0, 0.0).astype(jnp.float32)
                ns = 1.0 - samef
                o1 = ns * sk
                o2 = ns * (1.0 - sk) * lm
                o3 = ns * (1.0 - sk) * (1.0 - lm)
                relx = pxd - pxs; rely = pyd - pys
                d2 = relx * relx + rely * rely + 1e-12
                for r, val in enumerate((samef, o1, o2, o3, relx, rely, d2, samef)):
                    v_st[r, sl] = val
                return c2

            lax.fori_loop(0, _CHF // 16, group, 0)
            pltpu.sync_copy(v_st, out_h.at[:, pl.ds(base, _CHF)])
            return carry

        lax.fori_loop(0, nch, chunk, 0)

    return kfeat(src, dst, jt, px, py, skelf, limf)


def _sc_edge(xl, xr, ett, same, src, dst, attc):
    @functools.partial(
        pl.kernel, mesh=_sc_mesh(),
        compiler_params=pltpu.CompilerParams(needs_layout_passes=False),
        out_type=[jax.ShapeDtypeStruct((2 * _NPAD, 128), jnp.float32),
                  jax.ShapeDtypeStruct((2 * _DROWS, 128), jnp.float32)],
        scratch_types=[
            pltpu.VMEM((_CHE,), jnp.int32), pltpu.VMEM((_CHE,), jnp.int32),
            pltpu.VMEM((_CHE,), jnp.float32),
            pltpu.VMEM((_CHE, 128), jnp.float32), pltpu.VMEM((_CHE, 128), jnp.float32),
            pltpu.VMEM((_CHE, 128), jnp.float32),
            pltpu.VMEM((128,), jnp.float32),
            pltpu.VMEM((_CHE, 128), jnp.float32),
            pltpu.VMEM((16, 128), jnp.float32),
            pltpu.VMEM_SHARED((_NPAD, 128), jnp.float32),
            pltpu.VMEM_SHARED((_DROWS, 128), jnp.float32),
            pltpu.SemaphoreType.DMA, pltpu.SemaphoreType.DMA,
        ])
    def kedge(xl_h, xr_h, et_h, same_h, src_h, dst_h, att_h, num_h, den_h,
              v_src, v_dst, v_same, v_xl, v_xr, v_et, v_att, v_out,
              v_dstage, sp_num, sp_den, sem1, sem2):
        cid = lax.axis_index("c"); sid = lax.axis_index("s")
        wid = sid * 2 + cid
        zero16 = jnp.zeros((16,), jnp.float32)

        def zrow(r, c):
            for cc in range(8):
                v_out[r, pl.ds(cc * 16, 16)] = zero16
            return c

        lax.fori_loop(0, _CHE, zrow, 0)
        for r in range(16):
            for cc in range(8):
                v_dstage[r, pl.ds(cc * 16, 16)] = zero16

        for k in range(_RPT // _CHE):
            pltpu.sync_copy(v_out, sp_num.at[pl.ds(sid * _RPT + k * _CHE, _CHE)])
        pltpu.sync_copy(v_dstage, sp_den.at[pl.ds(sid * 20, 16)])
        pltpu.sync_copy(v_dstage.at[pl.ds(0, 4)], sp_den.at[pl.ds(sid * 20 + 16, 4)])
        plsc.subcore_barrier()
        pltpu.sync_copy(att_h, v_att)
        lidx = lax.iota(jnp.int32, 16)

        def chunk(it, carry):
            base = wid * _EPW + it * _CHE
            pltpu.sync_copy(src_h.at[pl.ds(base, _CHE)], v_src)
            pltpu.sync_copy(dst_h.at[pl.ds(base, _CHE)], v_dst)
            pltpu.sync_copy(same_h.at[pl.ds(base, _CHE)], v_same)
            pltpu.sync_copy(et_h.at[pl.ds(base, _CHE)], v_et)
            cp1 = pltpu.async_copy(xl_h.at[v_src], v_xl, sem1)
            cp2 = pltpu.async_copy(xr_h.at[v_dst], v_xr, sem2)
            cp1.wait(); cp2.wait()

            def group(g, c2):
                sl = pl.ds(g * 16, 16)
                rowidx = lidx + g * 16
                same16 = v_same[sl]
                dst16 = v_dst[sl]
                dcol0 = lax.bitwise_and(dst16, 31) * 4
                for h in range(4):
                    xls = []
                    acc = jnp.zeros((16,), jnp.float32)
                    for t in range(32):
                        j = h * 32 + t
                        cidx = _splat(j)
                        xlv = plsc.load_gather(v_xl, [rowidx, cidx])
                        xrv = plsc.load_gather(v_xr, [rowidx, cidx])
                        etv = plsc.load_gather(v_et, [rowidx, cidx])
                        av = plsc.load_gather(v_att, [cidx])
                        m = xlv + xrv + etv
                        ma = jnp.maximum(m, 0.2 * m)
                        acc = acc + ma * av
                        xls.append(xlv)
                    ex = jnp.exp(acc)
                    if h == 3:
                        ex = ex * same16
                    for t in range(32):
                        col = h * 32 + t if h < 3 else 96 + t
                        plsc.store_scatter(v_out, [rowidx, _splat(col)], ex * xls[t])
                    plsc.store_scatter(v_dstage, [lidx, dcol0 + h], ex)
                pltpu.sync_copy(v_dstage,
                                sp_den.at[lax.shift_right_logical(dst16, 5)],
                                add=True)
                for h in range(4):
                    plsc.store_scatter(v_dstage, [lidx, dcol0 + h], zero16)
                return c2

            lax.fori_loop(0, _CHE // 16, group, 0)
            pltpu.sync_copy(v_out, sp_num.at[v_dst], add=True)
            return carry

        lax.fori_loop(0, _NCHE, chunk, 0)
        plsc.subcore_barrier()
        pltpu.sync_copy(sp_num.at[pl.ds(sid * _RPT, _RPT)],
                        num_h.at[pl.ds(cid * _NPAD + sid * _RPT, _RPT)])

        @pl.when(sid < 2)
        def _():
            pltpu.sync_copy(sp_den.at[pl.ds(sid * (_DROWS // 2), _DROWS // 2)],
                            den_h.at[pl.ds(cid * _DROWS + sid * (_DROWS // 2),
                                           _DROWS // 2)])

    return kedge(xl, xr, ett, same, src, dst, attc)


_BN = 640


def _tc_dense(h, jtf, emb, wl, bl, wr, br, with_emb):
    def body(*refs):
        if with_emb:
            h_ref, jt_ref, emb_ref, wl_ref, bl_ref, wr_ref, br_ref, xl_ref, xr_ref = refs
            io = lax.broadcasted_iota(jnp.int32, (1, _NJT), 1).astype(jnp.float32)
            oh = jnp.where(jt_ref[...] == io, 1.0, 0.0).astype(jnp.float32)
            hh = h_ref[...] + jnp.dot(oh, emb_ref[...], preferred_element_type=jnp.float32)
        else:
            h_ref, wl_ref, bl_ref, wr_ref, br_ref, xl_ref, xr_ref = refs
            hh = h_ref[...]
        xl_ref[...] = jnp.dot(hh, wl_ref[...], preferred_element_type=jnp.float32) + bl_ref[...]
        xr_ref[...] = jnp.dot(hh, wr_ref[...], preferred_element_type=jnp.float32) + br_ref[...]

    din = h.shape[1]
    in_specs = [pl.BlockSpec((_BN, din), lambda i: (i, 0))]
    args = [h]
    if with_emb:
        in_specs += [pl.BlockSpec((_BN, 1), lambda i: (i, 0)),
                     pl.BlockSpec((_NJT, 128), lambda i: (0, 0))]
        args += [jtf, emb]
    in_specs += [pl.BlockSpec((din, 128), lambda i: (0, 0)),
                 pl.BlockSpec((1, 128), lambda i: (0, 0)),
                 pl.BlockSpec((din, 128), lambda i: (0, 0)),
                 pl.BlockSpec((1, 128), lambda i: (0, 0))]
    args += [wl, bl.reshape(1, -1), wr, br.reshape(1, -1)]
    return pl.pallas_call(
        body,
        grid=(_NPAD // _BN,),
        in_specs=in_specs,
        out_specs=[pl.BlockSpec((_BN, 128), lambda i: (i, 0)),
                   pl.BlockSpec((_BN, 128), lambda i: (i, 0))],
        out_shape=[jax.ShapeDtypeStruct((_NPAD, 128), jnp.float32),
                   jax.ShapeDtypeStruct((_NPAD, 128), jnp.float32)],
    )(*args)


_BE = 3200


def _tc_edgedense(erawt, w1, b1, w2, b2, wec):
    def body(f_ref, w1_ref, b1_ref, w2_ref, b2_ref, we_ref, o_ref):
        f = f_ref[...]
        fm = jnp.concatenate([f[0:6], jnp.sqrt(f[6:7]), f[7:8]], axis=0)
        ft = fm.T
        h1 = jnp.maximum(jnp.dot(ft, w1_ref[...], preferred_element_type=jnp.float32)
                         + b1_ref[...], 0.0)
        ea = jnp.dot(h1, w2_ref[...], preferred_element_type=jnp.float32) + b2_ref[...]
        o_ref[...] = jnp.dot(ea, we_ref[...], preferred_element_type=jnp.float32)

    return pl.pallas_call(
        body,
        grid=(_E // _BE,),
        in_specs=[pl.BlockSpec((8, _BE), lambda i: (0, i)),
                  pl.BlockSpec((8, 32), lambda i: (0, 0)),
                  pl.BlockSpec((1, 32), lambda i: (0, 0)),
                  pl.BlockSpec((32, 16), lambda i: (0, 0)),
                  pl.BlockSpec((1, 16), lambda i: (0, 0)),
                  pl.BlockSpec((16, 128), lambda i: (0, 0))],
        out_specs=pl.BlockSpec((_BE, 128), lambda i: (i, 0)),
        out_shape=jax.ShapeDtypeStruct((_E, 128), jnp.float32),
    )(erawt, w1, b1.reshape(1, -1), w2, b2.reshape(1, -1), wec)


def _ln_in(y, g, b):
    m = y.mean(-1, keepdims=True)
    v = ((y - m) ** 2).mean(-1, keepdims=True)
    return (y - m) / jnp.sqrt(v + 1e-5) * g + b


def _tc_node(num, den, biascat, g, b, last, projw=None, pb=None, fg=None, fb=None):
    def body(*refs):
        if last:
            n_ref, d_ref, bias_ref, g_ref, b_ref, pw_ref, pb_ref, fg_ref, fb_ref, o_ref = refs
        else:
            n_ref, d_ref, bias_ref, g_ref, b_ref, o_ref = refs
        p = n_ref[0] + n_ref[1]
        d = d_ref[0] + d_ref[1] + 1e-16
        parts = []
        for hh in range(3):
            parts.append(p[:, 32 * hh:32 * hh + 32] / d[:, hh:hh + 1])
        parts.append(p[:, 96:128] / d[:, 3:4])
        hcat = jnp.concatenate(parts, axis=1) + bias_ref[...]
        if last:
            hm = (hcat[:, 0:32] + hcat[:, 32:64] + hcat[:, 64:96] + hcat[:, 96:128]) * 0.25
            y = _ln_in(hm, g_ref[...], b_ref[...])
            he = jnp.where(y > 0, y, jnp.exp(y) - 1.0)
            z = jnp.dot(he, pw_ref[...], preferred_element_type=jnp.float32) + pb_ref[...]
            o_ref[...] = _ln_in(z, fg_ref[...], fb_ref[...])
        else:
            y = _ln_in(hcat, g_ref[...], b_ref[...])
            o_ref[...] = jnp.where(y > 0, y, jnp.exp(y) - 1.0)

    gdim = g.shape[0]
    in_specs = [pl.BlockSpec((2, _BN, 128), lambda i: (0, i, 0)),
                pl.BlockSpec((2, _BN, 4), lambda i: (0, i, 0)),
                pl.BlockSpec((1, 128), lambda i: (0, 0)),
                pl.BlockSpec((1, gdim), lambda i: (0, 0)),
                pl.BlockSpec((1, gdim), lambda i: (0, 0))]
    args = [num, den, biascat.reshape(1, -1), g.reshape(1, -1), b.reshape(1, -1)]
    if last:
        in_specs += [pl.BlockSpec((32, 128), lambda i: (0, 0)),
                     pl.BlockSpec((1, 128), lambda i: (0, 0)),
                     pl.BlockSpec((1, 128), lambda i: (0, 0)),
                     pl.BlockSpec((1, 128), lambda i: (0, 0))]
        args += [projw, pb.reshape(1, -1), fg.reshape(1, -1), fb.reshape(1, -1)]
    return pl.pallas_call(
        body,
        grid=(_NPAD // _BN,),
        in_specs=in_specs,
        out_specs=pl.BlockSpec((_BN, 128), lambda i: (i, 0)),
        out_shape=jax.ShapeDtypeStruct((_NPAD, 128), jnp.float32),
    )(*args)


def kernel(x, edge_index, joint_types, positions, params):
    src = edge_index[0].astype(jnp.int32)
    dst = edge_index[1].astype(jnp.int32)
    jt = joint_types.astype(jnp.int32)
    px = positions[:, 0]
    py = positions[:, 1]
    skelf = jnp.asarray(_SKEL_F)
    limf = jnp.asarray(_SLIMB_F)

    erawt = _sc_edgefeat(src, dst, jt, px, py, skelf, limf)
    same = erawt[0]

    xpad = jnp.pad(x, ((0, _NPAD - _N), (0, 0)))
    jtfpad = jnp.pad(jt.astype(jnp.float32)[:, None], ((0, _NPAD - _N), (0, 0)))

    h = xpad
    out = None
    for i, lp in enumerate(params['layers']):
        last = i == len(params['layers']) - 1
        wl = jnp.concatenate([lp['std']['Wl'], lp['rep']['Wl']], axis=1)
        bl = jnp.concatenate([lp['std']['bl'], lp['rep']['bl']])
        wr = jnp.concatenate([lp['std']['Wr'], lp['rep']['Wr']], axis=1)
        br = jnp.concatenate([lp['std']['br'], lp['rep']['br']])
        wec = jnp.concatenate([lp['std']['We'], lp['rep']['We']], axis=1)
        attc = jnp.concatenate([lp['std']['att'].reshape(-1), lp['rep']['att'].reshape(-1)])
        biascat = jnp.concatenate([lp['std']['bias'], lp['rep']['bias']])

        enc = lp['enc']
        w1p = jnp.concatenate([enc['W1'], jnp.zeros((1, 32), jnp.float32)], axis=0)

        xl, xr = _tc_dense(h, jtfpad, params['emb'], wl, bl, wr, br, with_emb=(i == 0))
        ett = _tc_edgedense(erawt, w1p, enc['b1'], enc['W2'], enc['b2'], wec)
        numflat, denflat = _sc_edge(xl, xr, ett, same, src, dst, attc)
        num = numflat.reshape(2, _NPAD, 128)
        den = denflat.reshape(2, _NPAD, 4)
        npar = params['norms'][i]
        if last:
            out = _tc_node(num, den, biascat, npar['g'], npar['b'], True,
                           params['proj_W'], params['proj_b'],
                           params['final_g'], params['final_b'])
        else:
            h = _tc_node(num, den, biascat, npar['g'], npar['b'], False)
    return out[:_N]


# speedup vs baseline: 24.6520x; 2.2071x over previous
import functools

import numpy as np
import jax
import jax.numpy as jnp
from jax import lax
from jax.experimental import pallas as pl
from jax.experimental.pallas import tpu as pltpu
from jax.experimental.pallas import tpu_sc as plsc

_COCO_SKELETON = [(0, 1), (0, 2), (1, 3), (2, 4), (5, 7), (7, 9), (6, 8), (8, 10), (5, 6), (5, 11), (6, 12), (11, 12), (11, 13), (13, 15), (12, 14), (14, 16)]
_LIMBS = [{0, 1, 2, 3, 4}, {5, 7, 9}, {6, 8, 10}, {5, 6, 11, 12}, {11, 13, 15}, {12, 14, 16}]
_NJT = 17
_N = 10000
_NPAD = 10240
_E = 320000
_NW = 32
_RPT = _NPAD // 16


_CHF = 128
_NCHUNKS = _E // _CHF
_CHE = 80
_EPW = _E // _NW
_NCHE = _EPW // _CHE
_DROWS = _NPAD * 4 // 128


def _build_mats():
    skel = np.zeros((_NJT, _NJT), dtype=bool)
    for a, b in _COCO_SKELETON:
        skel[a, b] = True; skel[b, a] = True
    slimb = np.zeros((_NJT, _NJT), dtype=bool)
    for limb in _LIMBS:
        for a in limb:
            for b in limb:
                if a != b and not skel[a, b]:
                    slimb[a, b] = True
    return skel, slimb


_SKEL_NP, _SLIMB_NP = _build_mats()
_SKEL_F = np.zeros((304,), np.float32); _SKEL_F[:289] = _SKEL_NP.astype(np.float32).reshape(-1)
_SLIMB_F = np.zeros((304,), np.float32); _SLIMB_F[:289] = _SLIMB_NP.astype(np.float32).reshape(-1)


def _sc_mesh():
    return plsc.VectorSubcoreMesh(core_axis_name="c", subcore_axis_name="s",
                                  num_cores=2, num_subcores=16)


def _splat(v, dtype=jnp.int32):
    return jnp.full((16,), v, dtype)


def _sc_edgefeat(src, dst, jt, px, py, skelf, limf):
    @functools.partial(
        pl.kernel, mesh=_sc_mesh(),
        compiler_params=pltpu.CompilerParams(needs_layout_passes=False),
        out_type=jax.ShapeDtypeStruct((8, _E), jnp.float32),
        scratch_types=[
            pltpu.VMEM((_CHF,), jnp.int32), pltpu.VMEM((_CHF,), jnp.int32),
            pltpu.VMEM((_N,), jnp.int32),
            pltpu.VMEM((_N,), jnp.float32), pltpu.VMEM((_N,), jnp.float32),
            pltpu.VMEM((304,), jnp.float32), pltpu.VMEM((304,), jnp.float32),
            pltpu.VMEM((8, _CHF), jnp.float32),
        ])
    def kfeat(src_h, dst_h, jt_h, px_h, py_h, skel_h, lim_h, out_h,
              v_src, v_dst, v_jt, v_px, v_py, v_sk, v_lm, v_st):
        cid = lax.axis_index("c"); sid = lax.axis_index("s")
        wid = sid * 2 + cid
        pltpu.sync_copy(jt_h, v_jt)
        pltpu.sync_copy(px_h, v_px)
        pltpu.sync_copy(py_h, v_py)
        pltpu.sync_copy(skel_h, v_sk)
        pltpu.sync_copy(lim_h, v_lm)
        rag = _NCHUNKS - (_NCHUNKS // _NW) * _NW
        nch = jnp.where(wid < rag, _NCHUNKS // _NW + 1, _NCHUNKS // _NW)

        def chunk(it, carry):
            base = (wid + _NW * it) * _CHF
            pltpu.sync_copy(src_h.at[pl.ds(base, _CHF)], v_src)
            pltpu.sync_copy(dst_h.at[pl.ds(base, _CHF)], v_dst)

            def group(g, c2):
                sl = pl.ds(g * 16, 16)
                s16 = v_src[sl]; d16 = v_dst[sl]
                jts = plsc.load_gather(v_jt, [s16])
                jtd = plsc.load_gather(v_jt, [d16])
                pxs = plsc.load_gather(v_px, [s16])
                pxd = plsc.load_gather(v_px, [d16])
                pys = plsc.load_gather(v_py, [s16])
                pyd = plsc.load_gather(v_py, [d16])
                ji = jts * 17 + jtd
                sk = plsc.load_gather(v_sk, [ji])
                lm = plsc.load_gather(v_lm, [ji])
                samef = jnp.where(jts == jtd, 1.0, 0.0).astype(jnp.float32)
                ns = 1.0 - samef
                o1 = ns * sk
                o2 = ns * (1.0 - sk) * lm
                o3 = ns * (1.0 - sk) * (1.0 - lm)
                relx = pxd - pxs; rely = pyd - pys
                d2 = relx * relx + rely * rely + 1e-12
                for r, val in enumerate((samef, o1, o2, o3, relx, rely, d2, samef)):
                    v_st[r, sl] = val
                return c2

            lax.fori_loop(0, _CHF // 16, group, 0)
            pltpu.sync_copy(v_st, out_h.at[:, pl.ds(base, _CHF)])
            return carry

        lax.fori_loop(0, nch, chunk, 0)

    return kfeat(src, dst, jt, px, py, skelf, limf)


def _sc_edge(xl, xr, ett, same, src, dst, attc):
    @functools.partial(
        pl.kernel, mesh=_sc_mesh(),
        compiler_params=pltpu.CompilerParams(needs_layout_passes=False),
        out_type=[jax.ShapeDtypeStruct((2 * _NPAD, 128), jnp.float32),
                  jax.ShapeDtypeStruct((2 * _DROWS, 128), jnp.float32)],
        scratch_types=[
            pltpu.VMEM((_CHE,), jnp.int32), pltpu.VMEM((_CHE,), jnp.int32),
            pltpu.VMEM((_CHE,), jnp.float32),
            pltpu.VMEM((_CHE, 128), jnp.float32), pltpu.VMEM((_CHE, 128), jnp.float32),
            pltpu.VMEM((_CHE, 128), jnp.float32),
            pltpu.VMEM((128,), jnp.float32),
            pltpu.VMEM((_CHE, 128), jnp.float32),
            pltpu.VMEM((16, 128), jnp.float32),
            pltpu.VMEM_SHARED((_NPAD, 128), jnp.float32),
            pltpu.VMEM_SHARED((_DROWS, 128), jnp.float32),
            pltpu.SemaphoreType.DMA, pltpu.SemaphoreType.DMA,
        ])
    def kedge(xl_h, xr_h, et_h, same_h, src_h, dst_h, att_h, num_h, den_h,
              v_src, v_dst, v_same, v_xl, v_xr, v_et, v_att, v_out,
              v_dstage, sp_num, sp_den, sem1, sem2):
        cid = lax.axis_index("c"); sid = lax.axis_index("s")
        wid = sid * 2 + cid
        zero16 = jnp.zeros((16,), jnp.float32)

        def zrow(r, c):
            for cc in range(8):
                v_out[r, pl.ds(cc * 16, 16)] = zero16
            return c

        lax.fori_loop(0, _CHE, zrow, 0)
        for r in range(16):
            for cc in range(8):
                v_dstage[r, pl.ds(cc * 16, 16)] = zero16

        for k in range(_RPT // _CHE):
            pltpu.sync_copy(v_out, sp_num.at[pl.ds(sid * _RPT + k * _CHE, _CHE)])
        pltpu.sync_copy(v_dstage, sp_den.at[pl.ds(sid * 20, 16)])
        pltpu.sync_copy(v_dstage.at[pl.ds(0, 4)], sp_den.at[pl.ds(sid * 20 + 16, 4)])
        plsc.subcore_barrier()
        pltpu.sync_copy(att_h, v_att)
        lidx = lax.iota(jnp.int32, 16)

        def chunk(it, carry):
            base = wid * _EPW + it * _CHE
            pltpu.sync_copy(src_h.at[pl.ds(base, _CHE)], v_src)
            pltpu.sync_copy(dst_h.at[pl.ds(base, _CHE)], v_dst)
            pltpu.sync_copy(same_h.at[pl.ds(base, _CHE)], v_same)
            pltpu.sync_copy(et_h.at[pl.ds(base, _CHE)], v_et)
            cp1 = pltpu.async_copy(xl_h.at[v_src], v_xl, sem1)
            cp2 = pltpu.async_copy(xr_h.at[v_dst], v_xr, sem2)
            cp1.wait(); cp2.wait()

            def group(g, c2):
                sl = pl.ds(g * 16, 16)
                rowidx = lidx + g * 16
                same16 = v_same[sl]
                dst16 = v_dst[sl]
                dcol0 = lax.bitwise_and(dst16, 31) * 4
                for h in range(4):
                    cbase = h * 32 if h < 3 else 96

                    def pass1(t, acc, cbase=cbase):
                        cidx = lax.bitwise_and(lidx + t, 31) + cbase
                        xlv = plsc.load_gather(v_xl, [rowidx, cidx])
                        xrv = plsc.load_gather(v_xr, [rowidx, cidx])
                        etv = plsc.load_gather(v_et, [rowidx, cidx])
                        av = plsc.load_gather(v_att, [cidx])
                        m = xlv + xrv + etv
                        ma = jnp.maximum(m, 0.2 * m)
                        plsc.store_scatter(v_out, [rowidx, cidx], xlv)
                        return acc + ma * av

                    acc = lax.fori_loop(0, 32, pass1, jnp.zeros((16,), jnp.float32))
                    ex = jnp.exp(acc)
                    if h == 3:
                        ex = ex * same16

                    def pass2(t, c, cbase=cbase, ex=ex):
                        cidx = lax.bitwise_and(lidx + t, 31) + cbase
                        xlv = plsc.load_gather(v_out, [rowidx, cidx])
                        plsc.store_scatter(v_out, [rowidx, cidx], ex * xlv)
                        return c

                    lax.fori_loop(0, 32, pass2, 0)
                    plsc.store_scatter(v_dstage, [lidx, dcol0 + h], ex)
                pltpu.sync_copy(v_dstage,
                                sp_den.at[lax.shift_right_logical(dst16, 5)],
                                add=True)
                for h in range(4):
                    plsc.store_scatter(v_dstage, [lidx, dcol0 + h], zero16)
                return c2

            lax.fori_loop(0, _CHE // 16, group, 0)
            pltpu.sync_copy(v_out, sp_num.at[v_dst], add=True)
            return carry

        lax.fori_loop(0, _NCHE, chunk, 0)
        plsc.subcore_barrier()
        pltpu.sync_copy(sp_num.at[pl.ds(sid * _RPT, _RPT)],
                        num_h.at[pl.ds(cid * _NPAD + sid * _RPT, _RPT)])

        @pl.when(sid < 2)
        def _():
            pltpu.sync_copy(sp_den.at[pl.ds(sid * (_DROWS // 2), _DROWS // 2)],
                            den_h.at[pl.ds(cid * _DROWS + sid * (_DROWS // 2),
                                           _DROWS // 2)])

    return kedge(xl, xr, ett, same, src, dst, attc)


_BN = 640


def _tc_dense(h, jtf, emb, wl, bl, wr, br, with_emb):
    def body(*refs):
        if with_emb:
            h_ref, jt_ref, emb_ref, wl_ref, bl_ref, wr_ref, br_ref, xl_ref, xr_ref = refs
            io = lax.broadcasted_iota(jnp.int32, (1, _NJT), 1).astype(jnp.float32)
            oh = jnp.where(jt_ref[...] == io, 1.0, 0.0).astype(jnp.float32)
            hh = h_ref[...] + jnp.dot(oh, emb_ref[...], preferred_element_type=jnp.float32)
        else:
            h_ref, wl_ref, bl_ref, wr_ref, br_ref, xl_ref, xr_ref = refs
            hh = h_ref[...]
        xl_ref[...] = jnp.dot(hh, wl_ref[...], preferred_element_type=jnp.float32) + bl_ref[...]
        xr_ref[...] = jnp.dot(hh, wr_ref[...], preferred_element_type=jnp.float32) + br_ref[...]

    din = h.shape[1]
    in_specs = [pl.BlockSpec((_BN, din), lambda i: (i, 0))]
    args = [h]
    if with_emb:
        in_specs += [pl.BlockSpec((_BN, 1), lambda i: (i, 0)),
                     pl.BlockSpec((_NJT, 128), lambda i: (0, 0))]
        args += [jtf, emb]
    in_specs += [pl.BlockSpec((din, 128), lambda i: (0, 0)),
                 pl.BlockSpec((1, 128), lambda i: (0, 0)),
                 pl.BlockSpec((din, 128), lambda i: (0, 0)),
                 pl.BlockSpec((1, 128), lambda i: (0, 0))]
    args += [wl, bl.reshape(1, -1), wr, br.reshape(1, -1)]
    return pl.pallas_call(
        body,
        grid=(_NPAD // _BN,),
        in_specs=in_specs,
        out_specs=[pl.BlockSpec((_BN, 128), lambda i: (i, 0)),
                   pl.BlockSpec((_BN, 128), lambda i: (i, 0))],
        out_shape=[jax.ShapeDtypeStruct((_NPAD, 128), jnp.float32),
                   jax.ShapeDtypeStruct((_NPAD, 128), jnp.float32)],
    )(*args)


_BE = 3200


def _tc_edgedense(erawt, w1, b1, w2, b2, wec):
    def body(f_ref, w1_ref, b1_ref, w2_ref, b2_ref, we_ref, o_ref):
        f = f_ref[...]
        fm = jnp.concatenate([f[0:6], jnp.sqrt(f[6:7]), f[7:8]], axis=0)
        ft = fm.T
        h1 = jnp.maximum(jnp.dot(ft, w1_ref[...], preferred_element_type=jnp.float32)
                         + b1_ref[...], 0.0)
        ea = jnp.dot(h1, w2_ref[...], preferred_element_type=jnp.float32) + b2_ref[...]
        o_ref[...] = jnp.dot(ea, we_ref[...], preferred_element_type=jnp.float32)

    return pl.pallas_call(
        body,
        grid=(_E // _BE,),
        in_specs=[pl.BlockSpec((8, _BE), lambda i: (0, i)),
                  pl.BlockSpec((8, 32), lambda i: (0, 0)),
                  pl.BlockSpec((1, 32), lambda i: (0, 0)),
                  pl.BlockSpec((32, 16), lambda i: (0, 0)),
                  pl.BlockSpec((1, 16), lambda i: (0, 0)),
                  pl.BlockSpec((16, 128), lambda i: (0, 0))],
        out_specs=pl.BlockSpec((_BE, 128), lambda i: (i, 0)),
        out_shape=jax.ShapeDtypeStruct((_E, 128), jnp.float32),
    )(erawt, w1, b1.reshape(1, -1), w2, b2.reshape(1, -1), wec)


def _ln_in(y, g, b):
    m = y.mean(-1, keepdims=True)
    v = ((y - m) ** 2).mean(-1, keepdims=True)
    return (y - m) / jnp.sqrt(v + 1e-5) * g + b


def _tc_node(num, den, biascat, g, b, last, projw=None, pb=None, fg=None, fb=None):
    def body(*refs):
        if last:
            n_ref, d_ref, bias_ref, g_ref, b_ref, pw_ref, pb_ref, fg_ref, fb_ref, o_ref = refs
        else:
            n_ref, d_ref, bias_ref, g_ref, b_ref, o_ref = refs
        p = n_ref[0] + n_ref[1]
        d = d_ref[0] + d_ref[1] + 1e-16
        parts = []
        for hh in range(3):
            parts.append(p[:, 32 * hh:32 * hh + 32] / d[:, hh:hh + 1])
        parts.append(p[:, 96:128] / d[:, 3:4])
        hcat = jnp.concatenate(parts, axis=1) + bias_ref[...]
        if last:
            hm = (hcat[:, 0:32] + hcat[:, 32:64] + hcat[:, 64:96] + hcat[:, 96:128]) * 0.25
            y = _ln_in(hm, g_ref[...], b_ref[...])
            he = jnp.where(y > 0, y, jnp.exp(y) - 1.0)
            z = jnp.dot(he, pw_ref[...], preferred_element_type=jnp.float32) + pb_ref[...]
            o_ref[...] = _ln_in(z, fg_ref[...], fb_ref[...])
        else:
            y = _ln_in(hcat, g_ref[...], b_ref[...])
            o_ref[...] = jnp.where(y > 0, y, jnp.exp(y) - 1.0)

    gdim = g.shape[0]
    in_specs = [pl.BlockSpec((2, _BN, 128), lambda i: (0, i, 0)),
                pl.BlockSpec((2, _BN, 4), lambda i: (0, i, 0)),
                pl.BlockSpec((1, 128), lambda i: (0, 0)),
                pl.BlockSpec((1, gdim), lambda i: (0, 0)),
                pl.BlockSpec((1, gdim), lambda i: (0, 0))]
    args = [num, den, biascat.reshape(1, -1), g.reshape(1, -1), b.reshape(1, -1)]
    if last:
        in_specs += [pl.BlockSpec((32, 128), lambda i: (0, 0)),
                     pl.BlockSpec((1, 128), lambda i: (0, 0)),
                     pl.BlockSpec((1, 128), lambda i: (0, 0)),
                     pl.BlockSpec((1, 128), lambda i: (0, 0))]
        args += [projw, pb.reshape(1, -1), fg.reshape(1, -1), fb.reshape(1, -1)]
    return pl.pallas_call(
        body,
        grid=(_NPAD // _BN,),
        in_specs=in_specs,
        out_specs=pl.BlockSpec((_BN, 128), lambda i: (i, 0)),
        out_shape=jax.ShapeDtypeStruct((_NPAD, 128), jnp.float32),
    )(*args)


def kernel(x, edge_index, joint_types, positions, params):
    src = edge_index[0].astype(jnp.int32)
    dst = edge_index[1].astype(jnp.int32)
    jt = joint_types.astype(jnp.int32)
    px = positions[:, 0]
    py = positions[:, 1]
    skelf = jnp.asarray(_SKEL_F)
    limf = jnp.asarray(_SLIMB_F)

    erawt = _sc_edgefeat(src, dst, jt, px, py, skelf, limf)
    same = erawt[0]

    xpad = jnp.pad(x, ((0, _NPAD - _N), (0, 0)))
    jtfpad = jnp.pad(jt.astype(jnp.float32)[:, None], ((0, _NPAD - _N), (0, 0)))

    h = xpad
    out = None
    for i, lp in enumerate(params['layers']):
        last = i == len(params['layers']) - 1
        wl = jnp.concatenate([lp['std']['Wl'], lp['rep']['Wl']], axis=1)
        bl = jnp.concatenate([lp['std']['bl'], lp['rep']['bl']])
        wr = jnp.concatenate([lp['std']['Wr'], lp['rep']['Wr']], axis=1)
        br = jnp.concatenate([lp['std']['br'], lp['rep']['br']])
        wec = jnp.concatenate([lp['std']['We'], lp['rep']['We']], axis=1)
        attc = jnp.concatenate([lp['std']['att'].reshape(-1), lp['rep']['att'].reshape(-1)])
        biascat = jnp.concatenate([lp['std']['bias'], lp['rep']['bias']])

        enc = lp['enc']
        w1p = jnp.concatenate([enc['W1'], jnp.zeros((1, 32), jnp.float32)], axis=0)

        xl, xr = _tc_dense(h, jtfpad, params['emb'], wl, bl, wr, br, with_emb=(i == 0))
        ett = _tc_edgedense(erawt, w1p, enc['b1'], enc['W2'], enc['b2'], wec)
        numflat, denflat = _sc_edge(xl, xr, ett, same, src, dst, attc)
        num = numflat.reshape(2, _NPAD, 128)
        den = denflat.reshape(2, _NPAD, 4)
        npar = params['norms'][i]
        if last:
            out = _tc_node(num, den, biascat, npar['g'], npar['b'], True,
                           params['proj_W'], params['proj_b'],
                           params['final_g'], params['final_b'])
        else:
            h = _tc_node(num, den, biascat, npar['g'], npar['b'], False)
    return out[:_N]


# speedup vs baseline: 25.4787x; 1.0335x over previous
import functools

import numpy as np
import jax
import jax.numpy as jnp
from jax import lax
from jax.experimental import pallas as pl
from jax.experimental.pallas import tpu as pltpu
from jax.experimental.pallas import tpu_sc as plsc

_COCO_SKELETON = [(0, 1), (0, 2), (1, 3), (2, 4), (5, 7), (7, 9), (6, 8), (8, 10), (5, 6), (5, 11), (6, 12), (11, 12), (11, 13), (13, 15), (12, 14), (14, 16)]
_LIMBS = [{0, 1, 2, 3, 4}, {5, 7, 9}, {6, 8, 10}, {5, 6, 11, 12}, {11, 13, 15}, {12, 14, 16}]
_NJT = 17
_N = 10000
_NPAD = 10240
_E = 320000
_NW = 32
_RPT = _NPAD // 16


_CHF = 128
_NCHUNKS = _E // _CHF
_CHE = 80
_EPW = _E // _NW
_NCHE = _EPW // _CHE
_DROWS = _NPAD * 4 // 128


def _build_mats():
    skel = np.zeros((_NJT, _NJT), dtype=bool)
    for a, b in _COCO_SKELETON:
        skel[a, b] = True; skel[b, a] = True
    slimb = np.zeros((_NJT, _NJT), dtype=bool)
    for limb in _LIMBS:
        for a in limb:
            for b in limb:
                if a != b and not skel[a, b]:
                    slimb[a, b] = True
    return skel, slimb


_SKEL_NP, _SLIMB_NP = _build_mats()
_SKEL_F = np.zeros((304,), np.float32); _SKEL_F[:289] = _SKEL_NP.astype(np.float32).reshape(-1)
_SLIMB_F = np.zeros((304,), np.float32); _SLIMB_F[:289] = _SLIMB_NP.astype(np.float32).reshape(-1)


def _sc_mesh():
    return plsc.VectorSubcoreMesh(core_axis_name="c", subcore_axis_name="s",
                                  num_cores=2, num_subcores=16)


def _splat(v, dtype=jnp.int32):
    return jnp.full((16,), v, dtype)


def _sc_edgefeat(src, dst, jt, px, py, skelf, limf):
    @functools.partial(
        pl.kernel, mesh=_sc_mesh(),
        compiler_params=pltpu.CompilerParams(needs_layout_passes=False),
        out_type=jax.ShapeDtypeStruct((8, _E), jnp.float32),
        scratch_types=[
            pltpu.VMEM((_CHF,), jnp.int32), pltpu.VMEM((_CHF,), jnp.int32),
            pltpu.VMEM((_N,), jnp.int32),
            pltpu.VMEM((_N,), jnp.float32), pltpu.VMEM((_N,), jnp.float32),
            pltpu.VMEM((304,), jnp.float32), pltpu.VMEM((304,), jnp.float32),
            pltpu.VMEM((8, _CHF), jnp.float32),
        ])
    def kfeat(src_h, dst_h, jt_h, px_h, py_h, skel_h, lim_h, out_h,
              v_src, v_dst, v_jt, v_px, v_py, v_sk, v_lm, v_st):
        cid = lax.axis_index("c"); sid = lax.axis_index("s")
        wid = sid * 2 + cid
        pltpu.sync_copy(jt_h, v_jt)
        pltpu.sync_copy(px_h, v_px)
        pltpu.sync_copy(py_h, v_py)
        pltpu.sync_copy(skel_h, v_sk)
        pltpu.sync_copy(lim_h, v_lm)
        rag = _NCHUNKS - (_NCHUNKS // _NW) * _NW
        nch = jnp.where(wid < rag, _NCHUNKS // _NW + 1, _NCHUNKS // _NW)

        def chunk(it, carry):
            base = (wid + _NW * it) * _CHF
            pltpu.sync_copy(src_h.at[pl.ds(base, _CHF)], v_src)
            pltpu.sync_copy(dst_h.at[pl.ds(base, _CHF)], v_dst)

            def group(g, c2):
                sl = pl.ds(g * 16, 16)
                s16 = v_src[sl]; d16 = v_dst[sl]
                jts = plsc.load_gather(v_jt, [s16])
                jtd = plsc.load_gather(v_jt, [d16])
                pxs = plsc.load_gather(v_px, [s16])
                pxd = plsc.load_gather(v_px, [d16])
                pys = plsc.load_gather(v_py, [s16])
                pyd = plsc.load_gather(v_py, [d16])
                ji = jts * 17 + jtd
                sk = plsc.load_gather(v_sk, [ji])
                lm = plsc.load_gather(v_lm, [ji])
                samef = jnp.where(jts == jtd, 1.0, 0.0).astype(jnp.float32)
                ns = 1.0 - samef
                o1 = ns * sk
                o2 = ns * (1.0 - sk) * lm
                o3 = ns * (1.0 - sk) * (1.0 - lm)
                relx = pxd - pxs; rely = pyd - pys
                d2 = relx * relx + rely * rely + 1e-12
                for r, val in enumerate((samef, o1, o2, o3, relx, rely, d2, samef)):
                    v_st[r, sl] = val
                return c2

            lax.fori_loop(0, _CHF // 16, group, 0)
            pltpu.sync_copy(v_st, out_h.at[:, pl.ds(base, _CHF)])
            return carry

        lax.fori_loop(0, nch, chunk, 0)

    return kfeat(src, dst, jt, px, py, skelf, limf)


def _sc_edge(xl, xr, ett, same, src, dst, attc):
    @functools.partial(
        pl.kernel, mesh=_sc_mesh(),
        compiler_params=pltpu.CompilerParams(needs_layout_passes=False),
        out_type=[jax.ShapeDtypeStruct((2 * _NPAD, 128), jnp.float32),
                  jax.ShapeDtypeStruct((2 * _DROWS, 128), jnp.float32)],
        scratch_types=[
            pltpu.VMEM((_CHE,), jnp.int32), pltpu.VMEM((_CHE,), jnp.int32),
            pltpu.VMEM((_CHE,), jnp.float32),
            pltpu.VMEM((_CHE, 128), jnp.float32), pltpu.VMEM((_CHE, 128), jnp.float32),
            pltpu.VMEM((_CHE, 128), jnp.float32),
            pltpu.VMEM((128,), jnp.float32),
            pltpu.VMEM((_CHE, 128), jnp.float32),
            pltpu.VMEM((16, 128), jnp.float32),
            pltpu.VMEM_SHARED((_NPAD, 128), jnp.float32),
            pltpu.VMEM_SHARED((_DROWS, 128), jnp.float32),
            pltpu.SemaphoreType.DMA, pltpu.SemaphoreType.DMA,
        ])
    def kedge(xl_h, xr_h, et_h, same_h, src_h, dst_h, att_h, num_h, den_h,
              v_src, v_dst, v_same, v_xl, v_xr, v_et, v_att, v_out,
              v_dstage, sp_num, sp_den, sem1, sem2):
        cid = lax.axis_index("c"); sid = lax.axis_index("s")
        wid = sid * 2 + cid
        zero16 = jnp.zeros((16,), jnp.float32)

        def zrow(r, c):
            for cc in range(8):
                v_out[r, pl.ds(cc * 16, 16)] = zero16
            return c

        lax.fori_loop(0, _CHE, zrow, 0)
        for r in range(16):
            for cc in range(8):
                v_dstage[r, pl.ds(cc * 16, 16)] = zero16

        for k in range(_RPT // _CHE):
            pltpu.sync_copy(v_out, sp_num.at[pl.ds(sid * _RPT + k * _CHE, _CHE)])
        pltpu.sync_copy(v_dstage, sp_den.at[pl.ds(sid * 20, 16)])
        pltpu.sync_copy(v_dstage.at[pl.ds(0, 4)], sp_den.at[pl.ds(sid * 20 + 16, 4)])
        plsc.subcore_barrier()
        pltpu.sync_copy(att_h, v_att)
        lidx = lax.iota(jnp.int32, 16)

        def chunk(it, carry):
            base = wid * _EPW + it * _CHE
            pltpu.sync_copy(src_h.at[pl.ds(base, _CHE)], v_src)
            pltpu.sync_copy(dst_h.at[pl.ds(base, _CHE)], v_dst)
            pltpu.sync_copy(same_h.at[pl.ds(base, _CHE)], v_same)
            pltpu.sync_copy(et_h.at[pl.ds(base, _CHE)], v_et)
            cp1 = pltpu.async_copy(xl_h.at[v_src], v_xl, sem1)
            cp2 = pltpu.async_copy(xr_h.at[v_dst], v_xr, sem2)
            cp1.wait(); cp2.wait()

            def group(g, c2):
                sl = pl.ds(g * 16, 16)
                rowidx = lidx + g * 16
                same16 = v_same[sl]
                dst16 = v_dst[sl]
                dcol0 = lax.bitwise_and(dst16, 31) * 4
                for h in range(4):
                    cbase = h * 32 if h < 3 else 96

                    def pass1(t, acc, cbase=cbase):
                        cidx = lax.bitwise_and(lidx + t, 31) + cbase
                        xlv = plsc.load_gather(v_xl, [rowidx, cidx])
                        xrv = plsc.load_gather(v_xr, [rowidx, cidx])
                        etv = plsc.load_gather(v_et, [rowidx, cidx])
                        av = plsc.load_gather(v_att, [cidx])
                        m = xlv + xrv + etv
                        ma = jnp.maximum(m, 0.2 * m)
                        plsc.store_scatter(v_out, [rowidx, cidx], xlv)
                        return acc + ma * av

                    acc = lax.fori_loop(0, 32, pass1, jnp.zeros((16,), jnp.float32), unroll=4)
                    ex = jnp.exp(acc)
                    if h == 3:
                        ex = ex * same16

                    def pass2(t, c, cbase=cbase, ex=ex):
                        cidx = lax.bitwise_and(lidx + t, 31) + cbase
                        xlv = plsc.load_gather(v_out, [rowidx, cidx])
                        plsc.store_scatter(v_out, [rowidx, cidx], ex * xlv)
                        return c

                    lax.fori_loop(0, 32, pass2, 0, unroll=4)
                    plsc.store_scatter(v_dstage, [lidx, dcol0 + h], ex)
                pltpu.sync_copy(v_dstage,
                                sp_den.at[lax.shift_right_logical(dst16, 5)],
                                add=True)
                for h in range(4):
                    plsc.store_scatter(v_dstage, [lidx, dcol0 + h], zero16)
                return c2

            lax.fori_loop(0, _CHE // 16, group, 0)
            pltpu.sync_copy(v_out, sp_num.at[v_dst], add=True)
            return carry

        lax.fori_loop(0, _NCHE, chunk, 0)
        plsc.subcore_barrier()
        pltpu.sync_copy(sp_num.at[pl.ds(sid * _RPT, _RPT)],
                        num_h.at[pl.ds(cid * _NPAD + sid * _RPT, _RPT)])

        @pl.when(sid < 2)
        def _():
            pltpu.sync_copy(sp_den.at[pl.ds(sid * (_DROWS // 2), _DROWS // 2)],
                            den_h.at[pl.ds(cid * _DROWS + sid * (_DROWS // 2),
                                           _DROWS // 2)])

    return kedge(xl, xr, ett, same, src, dst, attc)


_BN = 640


def _tc_dense(h, jtf, emb, wl, bl, wr, br, with_emb):
    def body(*refs):
        if with_emb:
            h_ref, jt_ref, emb_ref, wl_ref, bl_ref, wr_ref, br_ref, xl_ref, xr_ref = refs
            io = lax.broadcasted_iota(jnp.int32, (1, _NJT), 1).astype(jnp.float32)
            oh = jnp.where(jt_ref[...] == io, 1.0, 0.0).astype(jnp.float32)
            hh = h_ref[...] + jnp.dot(oh, emb_ref[...], preferred_element_type=jnp.float32)
        else:
            h_ref, wl_ref, bl_ref, wr_ref, br_ref, xl_ref, xr_ref = refs
            hh = h_ref[...]
        xl_ref[...] = jnp.dot(hh, wl_ref[...], preferred_element_type=jnp.float32) + bl_ref[...]
        xr_ref[...] = jnp.dot(hh, wr_ref[...], preferred_element_type=jnp.float32) + br_ref[...]

    din = h.shape[1]
    in_specs = [pl.BlockSpec((_BN, din), lambda i: (i, 0))]
    args = [h]
    if with_emb:
        in_specs += [pl.BlockSpec((_BN, 1), lambda i: (i, 0)),
                     pl.BlockSpec((_NJT, 128), lambda i: (0, 0))]
        args += [jtf, emb]
    in_specs += [pl.BlockSpec((din, 128), lambda i: (0, 0)),
                 pl.BlockSpec((1, 128), lambda i: (0, 0)),
                 pl.BlockSpec((din, 128), lambda i: (0, 0)),
                 pl.BlockSpec((1, 128), lambda i: (0, 0))]
    args += [wl, bl.reshape(1, -1), wr, br.reshape(1, -1)]
    return pl.pallas_call(
        body,
        grid=(_NPAD // _BN,),
        in_specs=in_specs,
        out_specs=[pl.BlockSpec((_BN, 128), lambda i: (i, 0)),
                   pl.BlockSpec((_BN, 128), lambda i: (i, 0))],
        out_shape=[jax.ShapeDtypeStruct((_NPAD, 128), jnp.float32),
                   jax.ShapeDtypeStruct((_NPAD, 128), jnp.float32)],
    )(*args)


_BE = 3200


def _tc_edgedense(erawt, w1, b1, w2, b2, wec):
    def body(f_ref, w1_ref, b1_ref, w2_ref, b2_ref, we_ref, o_ref):
        f = f_ref[...]
        fm = jnp.concatenate([f[0:6], jnp.sqrt(f[6:7]), f[7:8]], axis=0)
        ft = fm.T
        h1 = jnp.maximum(jnp.dot(ft, w1_ref[...], preferred_element_type=jnp.float32)
                         + b1_ref[...], 0.0)
        ea = jnp.dot(h1, w2_ref[...], preferred_element_type=jnp.float32) + b2_ref[...]
        o_ref[...] = jnp.dot(ea, we_ref[...], preferred_element_type=jnp.float32)

    return pl.pallas_call(
        body,
        grid=(_E // _BE,),
        in_specs=[pl.BlockSpec((8, _BE), lambda i: (0, i)),
                  pl.BlockSpec((8, 32), lambda i: (0, 0)),
                  pl.BlockSpec((1, 32), lambda i: (0, 0)),
                  pl.BlockSpec((32, 16), lambda i: (0, 0)),
                  pl.BlockSpec((1, 16), lambda i: (0, 0)),
                  pl.BlockSpec((16, 128), lambda i: (0, 0))],
        out_specs=pl.BlockSpec((_BE, 128), lambda i: (i, 0)),
        out_shape=jax.ShapeDtypeStruct((_E, 128), jnp.float32),
    )(erawt, w1, b1.reshape(1, -1), w2, b2.reshape(1, -1), wec)


def _ln_in(y, g, b):
    m = y.mean(-1, keepdims=True)
    v = ((y - m) ** 2).mean(-1, keepdims=True)
    return (y - m) / jnp.sqrt(v + 1e-5) * g + b


def _tc_node(num, den, biascat, g, b, last, projw=None, pb=None, fg=None, fb=None):
    def body(*refs):
        if last:
            n_ref, d_ref, bias_ref, g_ref, b_ref, pw_ref, pb_ref, fg_ref, fb_ref, o_ref = refs
        else:
            n_ref, d_ref, bias_ref, g_ref, b_ref, o_ref = refs
        p = n_ref[0] + n_ref[1]
        d = d_ref[0] + d_ref[1] + 1e-16
        parts = []
        for hh in range(3):
            parts.append(p[:, 32 * hh:32 * hh + 32] / d[:, hh:hh + 1])
        parts.append(p[:, 96:128] / d[:, 3:4])
        hcat = jnp.concatenate(parts, axis=1) + bias_ref[...]
        if last:
            hm = (hcat[:, 0:32] + hcat[:, 32:64] + hcat[:, 64:96] + hcat[:, 96:128]) * 0.25
            y = _ln_in(hm, g_ref[...], b_ref[...])
            he = jnp.where(y > 0, y, jnp.exp(y) - 1.0)
            z = jnp.dot(he, pw_ref[...], preferred_element_type=jnp.float32) + pb_ref[...]
            o_ref[...] = _ln_in(z, fg_ref[...], fb_ref[...])
        else:
            y = _ln_in(hcat, g_ref[...], b_ref[...])
            o_ref[...] = jnp.where(y > 0, y, jnp.exp(y) - 1.0)

    gdim = g.shape[0]
    in_specs = [pl.BlockSpec((2, _BN, 128), lambda i: (0, i, 0)),
                pl.BlockSpec((2, _BN, 4), lambda i: (0, i, 0)),
                pl.BlockSpec((1, 128), lambda i: (0, 0)),
                pl.BlockSpec((1, gdim), lambda i: (0, 0)),
                pl.BlockSpec((1, gdim), lambda i: (0, 0))]
    args = [num, den, biascat.reshape(1, -1), g.reshape(1, -1), b.reshape(1, -1)]
    if last:
        in_specs += [pl.BlockSpec((32, 128), lambda i: (0, 0)),
                     pl.BlockSpec((1, 128), lambda i: (0, 0)),
                     pl.BlockSpec((1, 128), lambda i: (0, 0)),
                     pl.BlockSpec((1, 128), lambda i: (0, 0))]
        args += [projw, pb.reshape(1, -1), fg.reshape(1, -1), fb.reshape(1, -1)]
    return pl.pallas_call(
        body,
        grid=(_NPAD // _BN,),
        in_specs=in_specs,
        out_specs=pl.BlockSpec((_BN, 128), lambda i: (i, 0)),
        out_shape=jax.ShapeDtypeStruct((_NPAD, 128), jnp.float32),
    )(*args)


def kernel(x, edge_index, joint_types, positions, params):
    src = edge_index[0].astype(jnp.int32)
    dst = edge_index[1].astype(jnp.int32)
    jt = joint_types.astype(jnp.int32)
    px = positions[:, 0]
    py = positions[:, 1]
    skelf = jnp.asarray(_SKEL_F)
    limf = jnp.asarray(_SLIMB_F)

    erawt = _sc_edgefeat(src, dst, jt, px, py, skelf, limf)
    same = erawt[0]

    xpad = jnp.pad(x, ((0, _NPAD - _N), (0, 0)))
    jtfpad = jnp.pad(jt.astype(jnp.float32)[:, None], ((0, _NPAD - _N), (0, 0)))

    h = xpad
    out = None
    for i, lp in enumerate(params['layers']):
        last = i == len(params['layers']) - 1
        wl = jnp.concatenate([lp['std']['Wl'], lp['rep']['Wl']], axis=1)
        bl = jnp.concatenate([lp['std']['bl'], lp['rep']['bl']])
        wr = jnp.concatenate([lp['std']['Wr'], lp['rep']['Wr']], axis=1)
        br = jnp.concatenate([lp['std']['br'], lp['rep']['br']])
        wec = jnp.concatenate([lp['std']['We'], lp['rep']['We']], axis=1)
        attc = jnp.concatenate([lp['std']['att'].reshape(-1), lp['rep']['att'].reshape(-1)])
        biascat = jnp.concatenate([lp['std']['bias'], lp['rep']['bias']])

        enc = lp['enc']
        w1p = jnp.concatenate([enc['W1'], jnp.zeros((1, 32), jnp.float32)], axis=0)

        xl, xr = _tc_dense(h, jtfpad, params['emb'], wl, bl, wr, br, with_emb=(i == 0))
        ett = _tc_edgedense(erawt, w1p, enc['b1'], enc['W2'], enc['b2'], wec)
        numflat, denflat = _sc_edge(xl, xr, ett, same, src, dst, attc)
        num = numflat.reshape(2, _NPAD, 128)
        den = denflat.reshape(2, _NPAD, 4)
        npar = params['norms'][i]
        if last:
            out = _tc_node(num, den, biascat, npar['g'], npar['b'], True,
                           params['proj_W'], params['proj_b'],
                           params['final_g'], params['final_b'])
        else:
            h = _tc_node(num, den, biascat, npar['g'], npar['b'], False)
    return out[:_N]
